# baseline (speedup 1.0000x reference)
from functools import partial

import jax
import jax.numpy as jnp
from jax import lax
from jax.experimental import pallas as pl
from jax.experimental.pallas import tpu as pltpu

H = 128
SUB = 8
LPAD = 32
NC = 128


def _lstm_kernel(ids_ref, lens_ref, table_ref, wih_ref, whh_ref, b_ref,
                 out_ref, xg_ref, *, batch):
    B = batch

    iota = lax.broadcasted_iota(jnp.int32, (1, NC), 1)
    onehot = jnp.where(ids_ref[...] == iota, 1.0, 0.0).astype(jnp.bfloat16)
    x = jnp.dot(onehot, table_ref[...],
                preferred_element_type=jnp.float32).astype(jnp.bfloat16)

    xg_ref[...] = (jnp.dot(x, wih_ref[...], preferred_element_type=jnp.float32)
                   + b_ref[...])

    lens = lens_ref[...]

    def step(t, carry):
        h, c = carry
        gates = xg_ref[pl.ds(t * B, B), :] + jnp.dot(
            h.astype(jnp.bfloat16), whh_ref[...],
            preferred_element_type=jnp.float32)
        sig = 0.5 * jnp.tanh(0.5 * gates[:, :3 * H]) + 0.5
        g = jnp.tanh(gates[:, 3 * H:])
        i, f, o = sig[:, :H], sig[:, H:2 * H], sig[:, 2 * H:]
        c_new = f * c + i * g
        h_new = o * jnp.tanh(c_new)
        valid = t < lens
        return jnp.where(valid, h_new, h), jnp.where(valid, c_new, c)

    h0 = jnp.zeros((B, H), jnp.float32)
    c0 = jnp.zeros((B, H), jnp.float32)
    h, _ = lax.fori_loop(0, LPAD, step, (h0, c0), unroll=True)
    out_ref[...] = h


@partial(jax.jit, static_argnames=("bg",))
def _run(ids_t, lens_t, table, wih, whh, b, *, bg):
    GB = ids_t.shape[0]
    B = bg * SUB
    rows = LPAD * B
    return pl.pallas_call(
        partial(_lstm_kernel, batch=B),
        grid=(GB,),
        in_specs=[
            pl.BlockSpec((None, rows, 1), lambda g: (g, 0, 0)),
            pl.BlockSpec((None, B, 1), lambda g: (g, 0, 0)),
            pl.BlockSpec((NC, H), lambda g: (0, 0)),
            pl.BlockSpec((H, 4 * H), lambda g: (0, 0)),
            pl.BlockSpec((H, 4 * H), lambda g: (0, 0)),
            pl.BlockSpec((1, 4 * H), lambda g: (0, 0)),
        ],
        out_specs=pl.BlockSpec((None, B, H), lambda g: (g, 0, 0)),
        out_shape=jax.ShapeDtypeStruct((GB, B, H), jnp.float32),
        scratch_shapes=[pltpu.VMEM((rows, 4 * H), jnp.float32)],
        compiler_params=pltpu.CompilerParams(
            dimension_semantics=("parallel",)),
    )(ids_t, lens_t, table, wih, whh, b)


def kernel(maxlen, ids, lens, table, wih, whh, b):
    G = ids.shape[0]
    bg = 32
    while G % bg:
        bg //= 2
    GB = G // bg
    B = bg * SUB
    ids_t = (ids.reshape(GB, bg, LPAD, SUB)
                .transpose(0, 2, 1, 3)
                .reshape(GB, LPAD * B, 1))
    lens_t = lens.reshape(GB, B, 1)
    out = _run(ids_t, lens_t, table, wih, whh, b, bg=bg)
    return out.reshape(G, SUB, H)


# speedup vs baseline: 1.1234x; 1.1234x over previous
from functools import partial

import jax
import jax.numpy as jnp
from jax import lax
from jax.experimental import pallas as pl
from jax.experimental.pallas import tpu as pltpu

H = 128
SUB = 8
LPAD = 32
NC = 128


def _lstm_kernel(ids_ref, lens_ref, table_ref, wih_ref, whh_ref, b_ref,
                 out_ref, xg_ref, *, batch):
    B = batch

    iota = lax.broadcasted_iota(jnp.int32, (1, NC), 1)
    onehot = jnp.where(ids_ref[...] == iota, 1.0, 0.0).astype(jnp.bfloat16)
    x = jnp.dot(onehot, table_ref[...],
                preferred_element_type=jnp.float32).astype(jnp.bfloat16)

    xg_ref[...] = (jnp.dot(x, wih_ref[...], preferred_element_type=jnp.float32)
                   + b_ref[...])

    lens = lens_ref[...]

    def step(t, carry):
        h, c = carry
        gates = xg_ref[pl.ds(t * B, B), :] + jnp.dot(
            h.astype(jnp.bfloat16), whh_ref[...],
            preferred_element_type=jnp.float32)
        sig = jax.nn.sigmoid(gates[:, :3 * H])
        g = jnp.tanh(gates[:, 3 * H:])
        i, f, o = sig[:, :H], sig[:, H:2 * H], sig[:, 2 * H:]
        c_new = f * c + i * g
        h_new = o * jnp.tanh(c_new)
        valid = t < lens
        return jnp.where(valid, h_new, h), jnp.where(valid, c_new, c)

    h0 = jnp.zeros((B, H), jnp.float32)
    c0 = jnp.zeros((B, H), jnp.float32)
    h, _ = lax.fori_loop(0, LPAD, step, (h0, c0), unroll=True)
    out_ref[...] = h


@partial(jax.jit, static_argnames=("bg",))
def _run(ids_t, lens_t, table, wih, whh, b, *, bg):
    GB = ids_t.shape[0]
    B = bg * SUB
    rows = LPAD * B
    return pl.pallas_call(
        partial(_lstm_kernel, batch=B),
        grid=(GB,),
        in_specs=[
            pl.BlockSpec((None, rows, 1), lambda g: (g, 0, 0)),
            pl.BlockSpec((None, B, 1), lambda g: (g, 0, 0)),
            pl.BlockSpec((NC, H), lambda g: (0, 0)),
            pl.BlockSpec((H, 4 * H), lambda g: (0, 0)),
            pl.BlockSpec((H, 4 * H), lambda g: (0, 0)),
            pl.BlockSpec((1, 4 * H), lambda g: (0, 0)),
        ],
        out_specs=pl.BlockSpec((None, B, H), lambda g: (g, 0, 0)),
        out_shape=jax.ShapeDtypeStruct((GB, B, H), jnp.float32),
        scratch_shapes=[pltpu.VMEM((rows, 4 * H), jnp.float32)],
        compiler_params=pltpu.CompilerParams(
            dimension_semantics=("parallel",)),
    )(ids_t, lens_t, table, wih, whh, b)


def kernel(maxlen, ids, lens, table, wih, whh, b):
    G = ids.shape[0]
    bg = 32
    while G % bg:
        bg //= 2
    GB = G // bg
    B = bg * SUB
    ids_t = (ids.reshape(GB, bg, LPAD, SUB)
                .transpose(0, 2, 1, 3)
                .reshape(GB, LPAD * B, 1))
    lens_t = lens.reshape(GB, B, 1)
    out = _run(ids_t, lens_t, table, wih, whh, b, bg=bg)
    return out.reshape(G, SUB, H)


# speedup vs baseline: 1.1338x; 1.0092x over previous
from functools import partial

import jax
import jax.numpy as jnp
from jax import lax
from jax.experimental import pallas as pl
from jax.experimental.pallas import tpu as pltpu

H = 128
SUB = 8
LPAD = 32
NC = 128


def _lstm_kernel(ids_ref, lens_ref, table_ref, wih_ref, whh_ref, b_ref,
                 out_ref, xg_ref, *, batch, chains):
    B = batch
    BC = B // chains

    iota = lax.broadcasted_iota(jnp.int32, (1, NC), 1)
    onehot = jnp.where(ids_ref[...] == iota, 1.0, 0.0).astype(jnp.bfloat16)
    x = jnp.dot(onehot, table_ref[...],
                preferred_element_type=jnp.float32).astype(jnp.bfloat16)

    xg_ref[...] = (jnp.dot(x, wih_ref[...], preferred_element_type=jnp.float32)
                   + b_ref[...])

    lens = lens_ref[...]

    def step(t, carry):
        out = []
        for k in range(chains):
            h, c = carry[k]
            gates = xg_ref[pl.ds(t * B + k * BC, BC), :] + jnp.dot(
                h.astype(jnp.bfloat16), whh_ref[...],
                preferred_element_type=jnp.float32)
            sig = jax.nn.sigmoid(gates[:, :3 * H])
            g = jnp.tanh(gates[:, 3 * H:])
            i, f, o = sig[:, :H], sig[:, H:2 * H], sig[:, 2 * H:]
            c_new = f * c + i * g
            h_new = o * jnp.tanh(c_new)
            valid = t < lens[k * BC:(k + 1) * BC]
            out.append((jnp.where(valid, h_new, h), jnp.where(valid, c_new, c)))
        return tuple(out)

    init = tuple((jnp.zeros((BC, H), jnp.float32), jnp.zeros((BC, H), jnp.float32))
                 for _ in range(chains))
    final = lax.fori_loop(0, LPAD, step, init, unroll=True)
    for k in range(chains):
        out_ref[pl.ds(k * BC, BC), :] = final[k][0]


@partial(jax.jit, static_argnames=("bg",))
def _run(ids_t, lens_t, table, wih, whh, b, *, bg):
    GB = ids_t.shape[0]
    B = bg * SUB
    rows = LPAD * B
    return pl.pallas_call(
        partial(_lstm_kernel, batch=B, chains=2),
        grid=(GB,),
        in_specs=[
            pl.BlockSpec((None, rows, 1), lambda g: (g, 0, 0)),
            pl.BlockSpec((None, B, 1), lambda g: (g, 0, 0)),
            pl.BlockSpec((NC, H), lambda g: (0, 0)),
            pl.BlockSpec((H, 4 * H), lambda g: (0, 0)),
            pl.BlockSpec((H, 4 * H), lambda g: (0, 0)),
            pl.BlockSpec((1, 4 * H), lambda g: (0, 0)),
        ],
        out_specs=pl.BlockSpec((None, B, H), lambda g: (g, 0, 0)),
        out_shape=jax.ShapeDtypeStruct((GB, B, H), jnp.float32),
        scratch_shapes=[pltpu.VMEM((rows, 4 * H), jnp.float32)],
        compiler_params=pltpu.CompilerParams(
            dimension_semantics=("parallel",)),
    )(ids_t, lens_t, table, wih, whh, b)


def kernel(maxlen, ids, lens, table, wih, whh, b):
    G = ids.shape[0]
    bg = 32
    while G % bg:
        bg //= 2
    GB = G // bg
    B = bg * SUB
    ids_t = (ids.reshape(GB, bg, LPAD, SUB)
                .transpose(0, 2, 1, 3)
                .reshape(GB, LPAD * B, 1))
    lens_t = lens.reshape(GB, B, 1)
    out = _run(ids_t, lens_t, table, wih, whh, b, bg=bg)
    return out.reshape(G, SUB, H)


# speedup vs baseline: 1.1651x; 1.0276x over previous
from functools import partial

import jax
import jax.numpy as jnp
from jax import lax
from jax.experimental import pallas as pl
from jax.experimental.pallas import tpu as pltpu

H = 128
SUB = 8
LPAD = 32
NC = 128


def _lstm_kernel(ids_ref, lens_ref, table_ref, wih_ref, whh_ref, b_ref,
                 out_ref, xg_ref, *, bg, chains):
    B = bg * SUB
    BGC = bg // chains
    BC = BGC * SUB

    iota = lax.broadcasted_iota(jnp.int32, (1, NC), 1)
    ids = ids_ref[...].reshape(bg * LPAD * SUB, 1)
    onehot = jnp.where(ids == iota, 1.0, 0.0).astype(jnp.bfloat16)
    x = jnp.dot(onehot, table_ref[...],
                preferred_element_type=jnp.float32).astype(jnp.bfloat16)

    xg = jnp.dot(x, wih_ref[...], preferred_element_type=jnp.float32) + b_ref[...]
    xg_ref[...] = xg.reshape(bg, LPAD * SUB, 4 * H)

    lens = lens_ref[...].reshape(B, 1)

    def step(t, carry):
        out = []
        for k in range(chains):
            h, c = carry[k]
            xs = xg_ref[k * BGC:(k + 1) * BGC, pl.ds(t * SUB, SUB), :]
            gates = xs.reshape(BC, 4 * H) + jnp.dot(
                h.astype(jnp.bfloat16), whh_ref[...],
                preferred_element_type=jnp.float32)
            sig = jax.nn.sigmoid(gates[:, :3 * H])
            g = jnp.tanh(gates[:, 3 * H:])
            i, f, o = sig[:, :H], sig[:, H:2 * H], sig[:, 2 * H:]
            c_new = f * c + i * g
            h_new = o * jnp.tanh(c_new)
            valid = t < lens[k * BC:(k + 1) * BC]
            out.append((jnp.where(valid, h_new, h), jnp.where(valid, c_new, c)))
        return tuple(out)

    init = tuple((jnp.zeros((BC, H), jnp.float32), jnp.zeros((BC, H), jnp.float32))
                 for _ in range(chains))
    final = lax.fori_loop(0, LPAD, step, init, unroll=True)
    for k in range(chains):
        out_ref[k * BGC:(k + 1) * BGC, :, :] = final[k][0].reshape(BGC, SUB, H)


@partial(jax.jit, static_argnames=("bg",))
def _run(ids, lens, table, wih, whh, b, *, bg):
    G = ids.shape[0]
    GB = G // bg
    rows = LPAD * SUB
    return pl.pallas_call(
        partial(_lstm_kernel, bg=bg, chains=2),
        grid=(GB,),
        in_specs=[
            pl.BlockSpec((bg, rows, 1), lambda g: (g, 0, 0)),
            pl.BlockSpec((bg, SUB, 1), lambda g: (g, 0, 0)),
            pl.BlockSpec((NC, H), lambda g: (0, 0)),
            pl.BlockSpec((H, 4 * H), lambda g: (0, 0)),
            pl.BlockSpec((H, 4 * H), lambda g: (0, 0)),
            pl.BlockSpec((1, 4 * H), lambda g: (0, 0)),
        ],
        out_specs=pl.BlockSpec((bg, SUB, H), lambda g: (g, 0, 0)),
        out_shape=jax.ShapeDtypeStruct((G, SUB, H), jnp.float32),
        scratch_shapes=[pltpu.VMEM((bg, rows, 4 * H), jnp.float32)],
        compiler_params=pltpu.CompilerParams(
            dimension_semantics=("parallel",)),
    )(ids, lens, table, wih, whh, b)


def kernel(maxlen, ids, lens, table, wih, whh, b):
    G = ids.shape[0]
    bg = 32
    while G % bg:
        bg //= 2
    return _run(ids, lens, table, wih, whh, b, bg=bg)


# speedup vs baseline: 1.2950x; 1.1116x over previous
from functools import partial

import jax
import jax.numpy as jnp
from jax import lax
from jax.experimental import pallas as pl
from jax.experimental.pallas import tpu as pltpu

H = 128
SUB = 8
LPAD = 32
NC = 128


def _lstm_kernel(ids_ref, lens_ref, table_ref, wih_ref, whh_ref, b_ref,
                 out_ref, xg_ref, *, bg, chains):
    B = bg * SUB
    BGC = bg // chains
    BC = BGC * SUB

    tw = (jnp.dot(table_ref[...], wih_ref[...],
                  preferred_element_type=jnp.float32)
          + b_ref[...]).astype(jnp.bfloat16)

    iota = lax.broadcasted_iota(jnp.int32, (1, NC), 1)
    ids = ids_ref[...].reshape(bg * LPAD * SUB, 1)
    onehot = jnp.where(ids == iota, 1.0, 0.0).astype(jnp.bfloat16)
    xg = jnp.dot(onehot, tw, preferred_element_type=jnp.float32)
    xg_ref[...] = xg.reshape(bg, LPAD * SUB, 4 * H)

    lens = lens_ref[...].reshape(B, 1)

    def step(t, carry):
        out = []
        for k in range(chains):
            h, c, hout = carry[k]
            xs = xg_ref[k * BGC:(k + 1) * BGC, pl.ds(t * SUB, SUB), :]
            gates = xs.reshape(BC, 4 * H) + jnp.dot(
                h.astype(jnp.bfloat16), whh_ref[...],
                preferred_element_type=jnp.float32)
            sig = jax.nn.sigmoid(gates[:, :3 * H])
            g = jnp.tanh(gates[:, 3 * H:])
            i, f, o = sig[:, :H], sig[:, H:2 * H], sig[:, 2 * H:]
            c_new = f * c + i * g
            h_new = o * jnp.tanh(c_new)
            last = t == lens[k * BC:(k + 1) * BC] - 1
            out.append((h_new, c_new, jnp.where(last, h_new, hout)))
        return tuple(out)

    zeros = lambda: jnp.zeros((BC, H), jnp.float32)
    init = tuple((zeros(), zeros(), zeros()) for _ in range(chains))
    final = lax.fori_loop(0, LPAD, step, init, unroll=True)
    for k in range(chains):
        out_ref[k * BGC:(k + 1) * BGC, :, :] = final[k][2].reshape(BGC, SUB, H)


@partial(jax.jit, static_argnames=("bg",))
def _run(ids, lens, table, wih, whh, b, *, bg):
    G = ids.shape[0]
    GB = G // bg
    rows = LPAD * SUB
    return pl.pallas_call(
        partial(_lstm_kernel, bg=bg, chains=2),
        grid=(GB,),
        in_specs=[
            pl.BlockSpec((bg, rows, 1), lambda g: (g, 0, 0)),
            pl.BlockSpec((bg, SUB, 1), lambda g: (g, 0, 0)),
            pl.BlockSpec((NC, H), lambda g: (0, 0)),
            pl.BlockSpec((H, 4 * H), lambda g: (0, 0)),
            pl.BlockSpec((H, 4 * H), lambda g: (0, 0)),
            pl.BlockSpec((1, 4 * H), lambda g: (0, 0)),
        ],
        out_specs=pl.BlockSpec((bg, SUB, H), lambda g: (g, 0, 0)),
        out_shape=jax.ShapeDtypeStruct((G, SUB, H), jnp.float32),
        scratch_shapes=[pltpu.VMEM((bg, rows, 4 * H), jnp.float32)],
        compiler_params=pltpu.CompilerParams(
            dimension_semantics=("parallel",)),
    )(ids, lens, table, wih, whh, b)


def kernel(maxlen, ids, lens, table, wih, whh, b):
    G = ids.shape[0]
    bg = 32
    while G % bg:
        bg //= 2
    return _run(ids, lens, table, wih, whh, b, bg=bg)
